# baseline (speedup 1.0000x reference)
import jax
import jax.numpy as jnp
from jax.experimental import pallas as pl


def _pe_add_kernel(x_ref, pe_ref, o_ref):
    pe = pe_ref[...]
    o_ref[...] = x_ref[...] + jnp.concatenate([pe, pe, pe, pe], axis=1)


def kernel(x, pos_emb):
    S, B, D = x.shape
    BS = 256
    x2 = x.reshape(S, B * D)
    out2 = pl.pallas_call(
        _pe_add_kernel,
        grid=(S // BS,),
        in_specs=[
            pl.BlockSpec((BS, B * D), lambda i: (i, 0)),
            pl.BlockSpec((BS, D), lambda i: (i, 0)),
        ],
        out_specs=pl.BlockSpec((BS, B * D), lambda i: (i, 0)),
        out_shape=jax.ShapeDtypeStruct((S, B * D), x.dtype),
    )(x2, pos_emb[:S])
    return out2.reshape(S, B, D)


# speedup vs baseline: 3.9152x; 3.9152x over previous
import jax
import jax.numpy as jnp
from jax.experimental import pallas as pl


def _pe_add_kernel(x_ref, pe_ref, o_ref):
    o_ref[...] = x_ref[...] + pe_ref[...][:, None, :]


def kernel(x, pos_emb):
    S, B, D = x.shape
    BS = 128
    return pl.pallas_call(
        _pe_add_kernel,
        grid=(S // BS,),
        in_specs=[
            pl.BlockSpec((BS, B, D), lambda i: (i, 0, 0)),
            pl.BlockSpec((BS, D), lambda i: (i, 0)),
        ],
        out_specs=pl.BlockSpec((BS, B, D), lambda i: (i, 0, 0)),
        out_shape=jax.ShapeDtypeStruct((S, B, D), x.dtype),
    )(x, pos_emb[:S])
